# baseline (speedup 1.0000x reference)
import jax
import jax.numpy as jnp
from jax.experimental import pallas as pl
from jax.experimental.pallas import tpu as pltpu

_BOOK_SIZE = 1024
_LATENT = 256
_N_PTS = 576
_NG = 9
_GR = 64

_X_PIECES = {
    0: [(0, 0, 256)],
    1: [(0, 256, 512)],
    2: [(0, 512, 576), (1, 0, 192)],
    3: [(1, 192, 448)],
    4: [(1, 448, 576), (2, 0, 128)],
    5: [(2, 128, 384)],
    6: [(2, 384, 576), (3, 0, 64)],
    7: [(3, 64, 320)],
    8: [(3, 320, 576)],
}
_ZQ_PIECES = {
    0: [(0, 0, 256), (1, 0, 256), (2, 0, 64)],
    1: [(2, 64, 256), (3, 0, 256), (4, 0, 128)],
    2: [(4, 128, 256), (5, 0, 256), (6, 0, 192)],
    3: [(6, 192, 256), (7, 0, 256), (8, 0, 256)],
}


def _dot(a, b_mat, dims):
    return jax.lax.dot_general(a, b_mat, (dims, ((), ())),
                               preferred_element_type=jnp.float32)


_BB = 2


def _vq_kernel(pq_ref, ze_ref, book_ref, prob_ref, logp_ref, zq_ref):
    pq = pq_ref[0, 0]
    bk = book_ref[...]
    bb = jnp.sum(bk * bk, axis=1)[None, :]
    for b2 in range(_BB):
        _vq_batch(pq, bk, bb, b2, ze_ref, prob_ref, logp_ref, zq_ref)


def _vq_batch(pq, bk, bb, b2, ze_ref, prob_ref, logp_ref, zq_ref):
    zeb = ze_ref[b2]
    zt = jnp.transpose(zeb)
    zt4 = jnp.reshape(zt, (_GR, 4, _N_PTS))

    groups = []
    for i in range(_NG):
        parts = [zt4[:, d, q0:q1] for (d, q0, q1) in _X_PIECES[i]]
        g_i = parts[0] if len(parts) == 1 else jnp.concatenate(parts, axis=1)
        groups.append(g_i[:, None, :])
    x = jnp.reshape(jnp.concatenate(groups, axis=1), (_N_PTS, _LATENT))

    g = _dot(x, bk, ((1,), (1,)))
    xx = jnp.sum(x * x, axis=1, keepdims=True)
    dist = (xx + bb) - 2.0 * g
    logits = -dist * pq
    m = jnp.max(logits, axis=1, keepdims=True)
    e = jnp.exp(logits - m)
    s = jnp.sum(e, axis=1, keepdims=True)
    rows = pl.ds(_N_PTS * b2, _N_PTS)
    prob_ref[rows, :] = e * (1.0 / s)
    logp_ref[rows, :] = (logits - m) - jnp.log(s)

    iota_book = jax.lax.broadcasted_iota(jnp.int32, (_N_PTS, _BOOK_SIZE), 1)
    masked = jnp.where(logits == m, iota_book, _BOOK_SIZE)
    idx = jnp.min(masked, axis=1, keepdims=True)
    onehot = (iota_book == idx).astype(jnp.float32)
    zq_flat = _dot(onehot, bk, ((1,), (0,)))

    zq3 = jnp.reshape(zq_flat, (_GR, _NG, _LATENT))
    rows_d = []
    for d in range(4):
        row_d = jnp.concatenate(
            [zq3[:, i, c0:c1] for (i, c0, c1) in _ZQ_PIECES[d]], axis=1)
        rows_d.append(row_d[:, None, :])
    ztq = jnp.reshape(jnp.concatenate(rows_d, axis=1), (_LATENT, _N_PTS))
    zq_ref[b2] = jnp.transpose(ztq)


def kernel(ze, book, log_param_q, is_train):
    b, n_pts, latent_ndim = ze.shape
    param_q = 1.0 + jnp.exp(log_param_q)
    precision_q = 0.5 / jnp.maximum(param_q, 1e-10)
    pq_arr = jnp.reshape(precision_q, (1, 1))
    rows = b * n_pts
    prob, logp, zq = pl.pallas_call(
        _vq_kernel,
        grid=(b // _BB,),
        in_specs=[
            pl.BlockSpec((1, 1), lambda i: (0, 0)),
            pl.BlockSpec((_BB, _N_PTS, _LATENT), lambda i: (i, 0, 0)),
            pl.BlockSpec((_BOOK_SIZE, _LATENT), lambda i: (0, 0)),
        ],
        out_specs=[
            pl.BlockSpec((_BB * _N_PTS, _BOOK_SIZE), lambda i: (i, 0)),
            pl.BlockSpec((_BB * _N_PTS, _BOOK_SIZE), lambda i: (i, 0)),
            pl.BlockSpec((_BB, _N_PTS, _LATENT), lambda i: (i, 0, 0)),
        ],
        out_shape=[
            jax.ShapeDtypeStruct((rows, _BOOK_SIZE), jnp.float32),
            jax.ShapeDtypeStruct((rows, _BOOK_SIZE), jnp.float32),
            jax.ShapeDtypeStruct((b, n_pts, latent_ndim), jnp.float32),
        ],
        compiler_params=pltpu.CompilerParams(
            dimension_semantics=("arbitrary",),
        ),
    )(pq_arr, ze, book)
    prob = prob.reshape(b, n_pts, _BOOK_SIZE)
    logp = logp.reshape(b, n_pts, _BOOK_SIZE)
    return (zq, precision_q, prob, logp)


# speedup vs baseline: 1.0116x; 1.0116x over previous
import jax
import jax.numpy as jnp
from jax.experimental import pallas as pl
from jax.experimental.pallas import tpu as pltpu

_BOOK_SIZE = 1024
_LATENT = 256
_N_PTS = 576
_NG = 9
_GR = 64

_X_PIECES = {
    0: [(0, 0, 256)],
    1: [(0, 256, 512)],
    2: [(0, 512, 576), (1, 0, 192)],
    3: [(1, 192, 448)],
    4: [(1, 448, 576), (2, 0, 128)],
    5: [(2, 128, 384)],
    6: [(2, 384, 576), (3, 0, 64)],
    7: [(3, 64, 320)],
    8: [(3, 320, 576)],
}
_ZQ_PIECES = {
    0: [(0, 0, 256), (1, 0, 256), (2, 0, 64)],
    1: [(2, 64, 256), (3, 0, 256), (4, 0, 128)],
    2: [(4, 128, 256), (5, 0, 256), (6, 0, 192)],
    3: [(6, 192, 256), (7, 0, 256), (8, 0, 256)],
}


def _dot(a, b_mat, dims):
    return jax.lax.dot_general(a, b_mat, (dims, ((), ())),
                               preferred_element_type=jnp.float32)


def _vq_kernel(pq_ref, ze_ref, book_ref, prob_ref, logp_ref, zq_ref):
    pq = pq_ref[0, 0]
    bk = book_ref[...]
    bb = jnp.sum(bk * bk, axis=1)[None, :]

    zeb = ze_ref[0]
    zt = jnp.transpose(zeb)
    zt4 = jnp.reshape(zt, (_GR, 4, _N_PTS))

    groups = []
    for i in range(_NG):
        parts = [zt4[:, d, q0:q1] for (d, q0, q1) in _X_PIECES[i]]
        g_i = parts[0] if len(parts) == 1 else jnp.concatenate(parts, axis=1)
        groups.append(g_i[:, None, :])
    x = jnp.reshape(jnp.concatenate(groups, axis=1), (_N_PTS, _LATENT))

    g = _dot(x, bk, ((1,), (1,)))
    xx = jnp.sum(x * x, axis=1, keepdims=True)
    dist = (xx + bb) - 2.0 * g
    logits = -dist * pq
    m = jnp.max(logits, axis=1, keepdims=True)
    e = jnp.exp(logits - m)
    s = jnp.sum(e, axis=1, keepdims=True)
    prob_ref[...] = e / s
    logp_ref[...] = (logits - m) - jnp.log(s)

    iota_book = jax.lax.broadcasted_iota(jnp.int32, (_N_PTS, _BOOK_SIZE), 1)
    masked = jnp.where(logits == m, iota_book, _BOOK_SIZE)
    idx = jnp.min(masked, axis=1, keepdims=True)
    onehot = (iota_book == idx).astype(jnp.float32)
    zq_flat = _dot(onehot, bk, ((1,), (0,)))

    zq3 = jnp.reshape(zq_flat, (_GR, _NG, _LATENT))
    t_iota = jax.lax.broadcasted_iota(jnp.int32, (_GR, _LATENT), 0)
    l_iota = jax.lax.broadcasted_iota(jnp.int32, (_GR, _LATENT), 1)
    ztq = None
    for d in range(4):
        row_d = jnp.concatenate(
            [zq3[:, i, c0:c1] for (i, c0, c1) in _ZQ_PIECES[d]], axis=1)
        sel_d = (l_iota == 4 * t_iota + d).astype(jnp.float32)
        term = _dot(sel_d, row_d, ((0,), (0,)))
        ztq = term if ztq is None else ztq + term
    eye = (jax.lax.broadcasted_iota(jnp.int32, (_LATENT, _LATENT), 0)
           == jax.lax.broadcasted_iota(jnp.int32, (_LATENT, _LATENT), 1)
           ).astype(jnp.float32)
    zq_ref[0] = _dot(ztq, eye, ((0,), (0,)))


def kernel(ze, book, log_param_q, is_train):
    b, n_pts, latent_ndim = ze.shape
    param_q = 1.0 + jnp.exp(log_param_q)
    precision_q = 0.5 / jnp.maximum(param_q, 1e-10)
    pq_arr = jnp.reshape(precision_q, (1, 1))
    rows = b * n_pts
    prob, logp, zq = pl.pallas_call(
        _vq_kernel,
        grid=(b,),
        in_specs=[
            pl.BlockSpec((1, 1), lambda i: (0, 0)),
            pl.BlockSpec((1, _N_PTS, _LATENT), lambda i: (i, 0, 0)),
            pl.BlockSpec((_BOOK_SIZE, _LATENT), lambda i: (0, 0)),
        ],
        out_specs=[
            pl.BlockSpec((_N_PTS, _BOOK_SIZE), lambda i: (i, 0)),
            pl.BlockSpec((_N_PTS, _BOOK_SIZE), lambda i: (i, 0)),
            pl.BlockSpec((1, _N_PTS, _LATENT), lambda i: (i, 0, 0)),
        ],
        out_shape=[
            jax.ShapeDtypeStruct((rows, _BOOK_SIZE), jnp.float32),
            jax.ShapeDtypeStruct((rows, _BOOK_SIZE), jnp.float32),
            jax.ShapeDtypeStruct((b, n_pts, latent_ndim), jnp.float32),
        ],
        compiler_params=pltpu.CompilerParams(
            dimension_semantics=("arbitrary",),
        ),
    )(pq_arr, ze, book)
    prob = prob.reshape(b, n_pts, _BOOK_SIZE)
    logp = logp.reshape(b, n_pts, _BOOK_SIZE)
    return (zq, precision_q, prob, logp)


# speedup vs baseline: 1.0158x; 1.0042x over previous
import jax
import jax.numpy as jnp
from jax.experimental import pallas as pl
from jax.experimental.pallas import tpu as pltpu

_BOOK_SIZE = 1024
_LATENT = 256
_N_PTS = 576
_NG = 9
_GR = 64

_X_PIECES = {
    0: [(0, 0, 256)],
    1: [(0, 256, 512)],
    2: [(0, 512, 576), (1, 0, 192)],
    3: [(1, 192, 448)],
    4: [(1, 448, 576), (2, 0, 128)],
    5: [(2, 128, 384)],
    6: [(2, 384, 576), (3, 0, 64)],
    7: [(3, 64, 320)],
    8: [(3, 320, 576)],
}
_ZQ_PIECES = {
    0: [(0, 0, 256), (1, 0, 256), (2, 0, 64)],
    1: [(2, 64, 256), (3, 0, 256), (4, 0, 128)],
    2: [(4, 128, 256), (5, 0, 256), (6, 0, 192)],
    3: [(6, 192, 256), (7, 0, 256), (8, 0, 256)],
}


def _dot(a, b_mat, dims):
    return jax.lax.dot_general(a, b_mat, (dims, ((), ())),
                               preferred_element_type=jnp.float32)


def _vq_kernel(pq_ref, ze_ref, book_ref, prob_ref, logp_ref, zq_ref):
    pq = pq_ref[0, 0]
    bk = book_ref[...]
    bb = jnp.sum(bk * bk, axis=1)[None, :]

    zeb = ze_ref[0]
    zt = jnp.transpose(zeb)
    zt4 = jnp.reshape(zt, (_GR, 4, _N_PTS))

    groups = []
    for i in range(_NG):
        parts = [zt4[:, d, q0:q1] for (d, q0, q1) in _X_PIECES[i]]
        g_i = parts[0] if len(parts) == 1 else jnp.concatenate(parts, axis=1)
        groups.append(g_i[:, None, :])
    x = jnp.reshape(jnp.concatenate(groups, axis=1), (_N_PTS, _LATENT))

    g = _dot(x, bk, ((1,), (1,)))
    xx = jnp.sum(x * x, axis=1, keepdims=True)
    dist = (xx + bb) - 2.0 * g
    logits = -dist * pq
    m = jnp.max(logits, axis=1, keepdims=True)
    e = jnp.exp(logits - m)
    s = jnp.sum(e, axis=1, keepdims=True)
    prob_ref[...] = e * (1.0 / s)
    logp_ref[...] = (logits - m) - jnp.log(s)

    iota_book = jax.lax.broadcasted_iota(jnp.int32, (_N_PTS, _BOOK_SIZE), 1)
    masked = jnp.where(logits == m, iota_book, _BOOK_SIZE)
    idx = jnp.min(masked, axis=1, keepdims=True)
    onehot = (iota_book == idx).astype(jnp.float32)
    zq_flat = _dot(onehot, bk, ((1,), (0,)))

    zq3 = jnp.reshape(zq_flat, (_GR, _NG, _LATENT))
    rows_d = []
    for d in range(4):
        row_d = jnp.concatenate(
            [zq3[:, i, c0:c1] for (i, c0, c1) in _ZQ_PIECES[d]], axis=1)
        rows_d.append(row_d[:, None, :])
    ztq = jnp.reshape(jnp.concatenate(rows_d, axis=1), (_LATENT, _N_PTS))
    zq_ref[0] = jnp.transpose(ztq)


def kernel(ze, book, log_param_q, is_train):
    b, n_pts, latent_ndim = ze.shape
    param_q = 1.0 + jnp.exp(log_param_q)
    precision_q = 0.5 / jnp.maximum(param_q, 1e-10)
    pq_arr = jnp.reshape(precision_q, (1, 1))
    rows = b * n_pts
    prob, logp, zq = pl.pallas_call(
        _vq_kernel,
        grid=(b,),
        in_specs=[
            pl.BlockSpec((1, 1), lambda i: (0, 0)),
            pl.BlockSpec((1, _N_PTS, _LATENT), lambda i: (i, 0, 0)),
            pl.BlockSpec((_BOOK_SIZE, _LATENT), lambda i: (0, 0)),
        ],
        out_specs=[
            pl.BlockSpec((_N_PTS, _BOOK_SIZE), lambda i: (i, 0)),
            pl.BlockSpec((_N_PTS, _BOOK_SIZE), lambda i: (i, 0)),
            pl.BlockSpec((1, _N_PTS, _LATENT), lambda i: (i, 0, 0)),
        ],
        out_shape=[
            jax.ShapeDtypeStruct((rows, _BOOK_SIZE), jnp.float32),
            jax.ShapeDtypeStruct((rows, _BOOK_SIZE), jnp.float32),
            jax.ShapeDtypeStruct((b, n_pts, latent_ndim), jnp.float32),
        ],
        compiler_params=pltpu.CompilerParams(
            dimension_semantics=("arbitrary",),
        ),
    )(pq_arr, ze, book)
    prob = prob.reshape(b, n_pts, _BOOK_SIZE)
    logp = logp.reshape(b, n_pts, _BOOK_SIZE)
    return (zq, precision_q, prob, logp)
